# baseline (speedup 1.0000x reference)
import functools

import jax
import jax.numpy as jnp
from jax import lax
from jax.experimental import pallas as pl
from jax.experimental.pallas import tpu as pltpu
from jax.experimental.pallas import tpu_sc as plsc

_CK = 32


def _sc_body(t_hbm, i_hbm, a_hbm, e_hbm, out_hbm, ids_hbm, buf, bias_v, ids_v,
             *, B, H, seg_lens, tot, nw, nc):
    cid = lax.axis_index("c")
    sid = lax.axis_index("s")
    wid = sid * nc + cid

    off = 0
    for m, lm in enumerate(seg_lens):
        rm = B * lm // nw
        in_base = wid * rm
        b = in_base // lm
        l0 = in_base - b * lm
        out_base = b * tot + off + l0

        pltpu.sync_copy(e_hbm.at[m, :], bias_v)

        for i in range(rm // 16):
            ids_v[pl.ds(i * 16, 16)] = jnp.full((16,), m, jnp.int32)
        pltpu.sync_copy(ids_v.at[pl.ds(0, rm)], ids_hbm.at[pl.ds(out_base, rm)])

        def chunk(k, _, x_hbm=t_hbm if m == 0 else (i_hbm if m == 1 else a_hbm),
                  in_base=in_base, out_base=out_base):
            pltpu.sync_copy(x_hbm.at[pl.ds(in_base + k * _CK, _CK), :], buf)

            def row(r, _):
                for c in range(H // 16):
                    sl = pl.ds(c * 16, 16)
                    buf[r, sl] = buf[r, sl] + bias_v[sl]
                return 0

            lax.fori_loop(0, _CK, row, 0)
            pltpu.sync_copy(buf, out_hbm.at[pl.ds(out_base + k * _CK, _CK), :])
            return 0

        lax.fori_loop(0, rm // _CK, chunk, 0)
        off += lm


def kernel(text, image, audio, modality_embed):
    B, l_t, H = text.shape
    l_i = image.shape[1]
    l_a = audio.shape[1]
    tot = l_t + l_i + l_a

    info = plsc.get_sparse_core_info()
    nc, ns = info.num_cores, info.num_subcores
    nw = nc * ns
    mesh = plsc.VectorSubcoreMesh(core_axis_name="c", subcore_axis_name="s")

    body = functools.partial(_sc_body, B=B, H=H, seg_lens=(l_t, l_i, l_a),
                             tot=tot, nw=nw, nc=nc)

    sck = pl.kernel(
        body,
        mesh=mesh,
        out_type=[
            jax.ShapeDtypeStruct((B * tot, H), jnp.float32),
            jax.ShapeDtypeStruct((B * tot,), jnp.int32),
        ],
        scratch_types=[
            pltpu.VMEM((_CK, H), jnp.float32),
            pltpu.VMEM((H,), jnp.float32),
            pltpu.VMEM((B * l_t // nw,), jnp.int32),
        ],
    )
    out2, ids1 = sck(
        text.reshape(B * l_t, H),
        image.reshape(B * l_i, H),
        audio.reshape(B * l_a, H),
        modality_embed,
    )
    return out2.reshape(B, tot, H), ids1.reshape(B, tot)


# speedup vs baseline: 3.8497x; 3.8497x over previous
import functools

import jax
import jax.numpy as jnp
from jax import lax
from jax.experimental import pallas as pl
from jax.experimental.pallas import tpu as pltpu
from jax.experimental.pallas import tpu_sc as plsc

_C = 512


def _tc_body(t_ref, i_ref, a_ref, emb_ref, out_ref, *, n_t, n_i):
    j = pl.program_id(1)

    @pl.when(j < n_t)
    def _():
        out_ref[...] = t_ref[...] + emb_ref[0, :][None, None, :]

    @pl.when((j >= n_t) & (j < n_t + n_i))
    def _():
        out_ref[...] = i_ref[...] + emb_ref[1, :][None, None, :]

    @pl.when(j >= n_t + n_i)
    def _():
        out_ref[...] = a_ref[...] + emb_ref[2, :][None, None, :]


def _sc_ids_body(ids_hbm, ids_v, *, B, seg_lens, tot, nw, nc):
    cid = lax.axis_index("c")
    sid = lax.axis_index("s")
    wid = sid * nc + cid

    off = 0
    for m, lm in enumerate(seg_lens):
        rm = B * lm // nw
        in_base = wid * rm
        b = in_base // lm
        l0 = in_base - b * lm
        out_base = b * tot + off + l0
        for i in range(rm // 16):
            ids_v[pl.ds(i * 16, 16)] = jnp.full((16,), m, jnp.int32)
        pltpu.sync_copy(ids_v.at[pl.ds(0, rm)], ids_hbm.at[pl.ds(out_base, rm)])
        off += lm


def kernel(text, image, audio, modality_embed):
    B, l_t, H = text.shape
    l_i = image.shape[1]
    l_a = audio.shape[1]
    tot = l_t + l_i + l_a
    n_t, n_i, n_a = l_t // _C, l_i // _C, l_a // _C

    info = plsc.get_sparse_core_info()
    nc, ns = info.num_cores, info.num_subcores
    nw = nc * ns
    mesh = plsc.VectorSubcoreMesh(core_axis_name="c", subcore_axis_name="s")

    ids1 = pl.kernel(
        functools.partial(_sc_ids_body, B=B, seg_lens=(l_t, l_i, l_a),
                          tot=tot, nw=nw, nc=nc),
        mesh=mesh,
        out_type=[jax.ShapeDtypeStruct((B * tot,), jnp.int32)],
        scratch_types=[pltpu.VMEM((B * l_t // nw,), jnp.int32)],
    )()[0]

    out = pl.pallas_call(
        functools.partial(_tc_body, n_t=n_t, n_i=n_i),
        grid=(B, n_t + n_i + n_a),
        in_specs=[
            pl.BlockSpec((1, _C, H), lambda b, j: (b, jnp.minimum(j, n_t - 1), 0)),
            pl.BlockSpec((1, _C, H), lambda b, j: (b, jnp.clip(j - n_t, 0, n_i - 1), 0)),
            pl.BlockSpec((1, _C, H), lambda b, j: (b, jnp.clip(j - n_t - n_i, 0, n_a - 1), 0)),
            pl.BlockSpec(modality_embed.shape, lambda b, j: (0, 0)),
        ],
        out_specs=pl.BlockSpec((1, _C, H), lambda b, j: (b, j, 0)),
        out_shape=jax.ShapeDtypeStruct((B, tot, H), jnp.float32),
    )(text, image, audio, modality_embed)

    return out, ids1.reshape(B, tot)


# speedup vs baseline: 3.8667x; 1.0044x over previous
import functools

import jax
import jax.numpy as jnp
from jax import lax
from jax.experimental import pallas as pl
from jax.experimental.pallas import tpu as pltpu
from jax.experimental.pallas import tpu_sc as plsc

_C = 512


def _tc_body(t_ref, i_ref, a_ref, emb_ref, out_ref, *, n_t, n_i):
    j = pl.program_id(1)

    @pl.when(j < n_t)
    def _():
        out_ref[...] = t_ref[...] + emb_ref[0, :][None, None, :]

    @pl.when((j >= n_t) & (j < n_t + n_i))
    def _():
        out_ref[...] = i_ref[...] + emb_ref[1, :][None, None, :]

    @pl.when(j >= n_t + n_i)
    def _():
        out_ref[...] = a_ref[...] + emb_ref[2, :][None, None, :]


def _sc_ids_body(ids_hbm, ids_v, *, B, seg_lens, tot, nw, nc):
    cid = lax.axis_index("c")
    sid = lax.axis_index("s")
    wid = sid * nc + cid

    off = 0
    for m, lm in enumerate(seg_lens):
        rm = B * lm // nw
        in_base = wid * rm
        b = in_base // lm
        l0 = in_base - b * lm
        out_base = b * tot + off + l0
        for i in range(rm // 16):
            ids_v[pl.ds(i * 16, 16)] = jnp.full((16,), m, jnp.int32)
        pltpu.sync_copy(ids_v.at[pl.ds(0, rm)], ids_hbm.at[pl.ds(out_base, rm)])
        off += lm


def kernel(text, image, audio, modality_embed):
    B, l_t, H = text.shape
    l_i = image.shape[1]
    l_a = audio.shape[1]
    tot = l_t + l_i + l_a
    n_t, n_i, n_a = l_t // _C, l_i // _C, l_a // _C

    info = plsc.get_sparse_core_info()
    nc, ns = info.num_cores, info.num_subcores
    nw = nc * ns
    mesh = plsc.VectorSubcoreMesh(core_axis_name="c", subcore_axis_name="s")

    out = pl.pallas_call(
        functools.partial(_tc_body, n_t=n_t, n_i=n_i),
        grid=(B, n_t + n_i + n_a),
        in_specs=[
            pl.BlockSpec((1, _C, H), lambda b, j: (b, jnp.minimum(j, n_t - 1), 0)),
            pl.BlockSpec((1, _C, H), lambda b, j: (b, jnp.clip(j - n_t, 0, n_i - 1), 0)),
            pl.BlockSpec((1, _C, H), lambda b, j: (b, jnp.clip(j - n_t - n_i, 0, n_a - 1), 0)),
            pl.BlockSpec(modality_embed.shape, lambda b, j: (0, 0)),
        ],
        out_specs=pl.BlockSpec((1, _C, H), lambda b, j: (b, j, 0)),
        out_shape=jax.ShapeDtypeStruct((B, tot, H), jnp.float32),
    )(text, image, audio, modality_embed)

    ids1 = pl.kernel(
        functools.partial(_sc_ids_body, B=B, seg_lens=(l_t, l_i, l_a),
                          tot=tot, nw=nw, nc=nc),
        mesh=mesh,
        out_type=[jax.ShapeDtypeStruct((B * tot,), jnp.int32)],
        scratch_types=[pltpu.VMEM((B * l_t // nw,), jnp.int32)],
    )()[0]

    return out, ids1.reshape(B, tot)
